# baseline (speedup 1.0000x reference)
import functools

import jax
import jax.numpy as jnp
from jax import lax
from jax.experimental import pallas as pl
from jax.experimental.pallas import tpu as pltpu
from jax.experimental.pallas import tpu_sc as plsc

N = 10000
E = 160000
R = 8
NB = 4
D = 256

NC = 2
NS = 16
NW = NC * NS
EPW = E // NW
NSCAN = EPW // 16 + 1
BROWS = 313
ACC_R = 320
TRASH = 316
CAP = 512
CH = 16
CNTW = 64

NBLK = 10
BLK = N // NBLK

_SC_PARAMS = pltpu.CompilerParams(use_tc_tiling_on_sc=False,
                                  needs_layout_passes=False)


def _rel_weight(basis_ref, comp_ref, r):
    def b16(x):
        return x.astype(jnp.bfloat16).astype(jnp.float32)

    w = b16(comp_ref[r, 0]) * b16(basis_ref[0])
    for b in range(1, NB):
        w = w + b16(comp_ref[r, b]) * b16(basis_ref[b])
    return w


def _tc_first_body(x_ref, basis_ref, comp_ref, hall_ref):
    r = pl.program_id(1)
    w = _rel_weight(basis_ref, comp_ref, r)
    hall_ref[...] = jnp.dot(x_ref[...].astype(jnp.bfloat16),
                            w.astype(jnp.bfloat16),
                            preferred_element_type=jnp.float32)


def _tc_first(x, basis, comp):
    return pl.pallas_call(
        _tc_first_body,
        grid=(NBLK, R),
        in_specs=[
            pl.BlockSpec((BLK, D), lambda n, r: (n, 0)),
            pl.BlockSpec((NB, D, D), lambda n, r: (0, 0, 0)),
            pl.BlockSpec(memory_space=pltpu.SMEM),
        ],
        out_specs=pl.BlockSpec((BLK, D), lambda n, r: (n, r)),
        out_shape=jax.ShapeDtypeStruct((N, R * D), jnp.float32),
    )(x, basis, comp)


def _tc_mid_body(agg_ref, x_ref, lw_ref, b_ref, basis_ref, comp_ref,
                 h_ref, hall_ref, hs_ref):
    r = pl.program_id(1)

    @pl.when(r == 0)
    def _():
        t = agg_ref[...] + jnp.dot(x_ref[...].astype(jnp.bfloat16),
                                   lw_ref[...].astype(jnp.bfloat16),
                                   preferred_element_type=jnp.float32)
        t = jnp.maximum(t + b_ref[...], 0.0)
        hs_ref[...] = t
        h_ref[...] = t

    w = _rel_weight(basis_ref, comp_ref, r)
    hall_ref[...] = jnp.dot(hs_ref[...].astype(jnp.bfloat16),
                            w.astype(jnp.bfloat16),
                            preferred_element_type=jnp.float32)


def _tc_mid(agg, x, loop_w, bias, basis, comp):
    return pl.pallas_call(
        _tc_mid_body,
        grid=(NBLK, R),
        in_specs=[
            pl.BlockSpec((BLK, D), lambda n, r: (n, 0)),
            pl.BlockSpec((BLK, D), lambda n, r: (n, 0)),
            pl.BlockSpec((D, D), lambda n, r: (0, 0)),
            pl.BlockSpec((1, D), lambda n, r: (0, 0)),
            pl.BlockSpec((NB, D, D), lambda n, r: (0, 0, 0)),
            pl.BlockSpec(memory_space=pltpu.SMEM),
        ],
        out_specs=[
            pl.BlockSpec((BLK, D), lambda n, r: (n, 0)),
            pl.BlockSpec((BLK, D), lambda n, r: (n, r)),
        ],
        out_shape=[
            jax.ShapeDtypeStruct((N, D), jnp.float32),
            jax.ShapeDtypeStruct((N, R * D), jnp.float32),
        ],
        scratch_shapes=[pltpu.VMEM((BLK, D), jnp.float32)],
    )(agg, x, loop_w, bias, basis, comp)


def _tc_last_body(agg_ref, x_ref, lw_ref, b_ref, fcw_ref, fcb_ref, out_ref):
    t = agg_ref[...] + jnp.dot(x_ref[...].astype(jnp.bfloat16),
                               lw_ref[...].astype(jnp.bfloat16),
                               preferred_element_type=jnp.float32)
    h = jnp.maximum(t + b_ref[...], 0.0)
    t = jnp.dot(h.astype(jnp.bfloat16), fcw_ref[...].astype(jnp.bfloat16),
                preferred_element_type=jnp.float32)
    t = t + fcb_ref[...]
    m = jnp.max(t, axis=1, keepdims=True)
    e = jnp.exp(t - m)
    out_ref[...] = e / jnp.sum(e, axis=1, keepdims=True)


def _tc_last(agg, x, loop_w, bias, fc_w, fc_b):
    return pl.pallas_call(
        _tc_last_body,
        grid=(NBLK,),
        in_specs=[
            pl.BlockSpec((BLK, D), lambda n: (n, 0)),
            pl.BlockSpec((BLK, D), lambda n: (n, 0)),
            pl.BlockSpec((D, D), lambda n: (0, 0)),
            pl.BlockSpec((1, D), lambda n: (0, 0)),
            pl.BlockSpec((D, D), lambda n: (0, 0)),
            pl.BlockSpec((1, D), lambda n: (0, 0)),
        ],
        out_specs=pl.BlockSpec((BLK, D), lambda n: (n, 0)),
        out_shape=jax.ShapeDtypeStruct((N, D), jnp.float32),
    )(agg, x, loop_w, bias, fc_w, fc_b)


def _sc_mesh():
    return plsc.VectorSubcoreMesh(core_axis_name="c", subcore_axis_name="s",
                                  num_cores=NC, num_subcores=NS)


@functools.lru_cache(maxsize=None)
def _make_sc_route():
    return functools.partial(
        pl.kernel,
        mesh=_sc_mesh(),
        compiler_params=_SC_PARAMS,
        out_type=(jax.ShapeDtypeStruct((NW * NW * CAP,), jnp.int32),
                  jax.ShapeDtypeStruct((NW * CNTW,), jnp.int32)),
        scratch_types=[
            pltpu.VMEM((EPW + 16,), jnp.int32),
            pltpu.VMEM((EPW + 16,), jnp.int32),
            pltpu.VMEM((EPW + 16,), jnp.int32),
            pltpu.VMEM((NW * CAP,), jnp.int32),
            pltpu.VMEM((CNTW,), jnp.int32),
        ],
    )(_sc_route_body)


def _sc_route_body(src_hbm, dst_hbm, et_hbm, packed_hbm, cnt_hbm,
                   src_v, dst_v, et_v, lists_v, cnt_v):
    cid = lax.axis_index("c")
    sid = lax.axis_index("s")
    w = cid * NS + sid

    pltpu.sync_copy(src_hbm.at[pl.ds(w * EPW, EPW)], src_v.at[pl.ds(0, EPW)])
    pltpu.sync_copy(dst_hbm.at[pl.ds(w * EPW, EPW)], dst_v.at[pl.ds(0, EPW)])
    pltpu.sync_copy(et_hbm.at[pl.ds(w * EPW, EPW)], et_v.at[pl.ds(0, EPW)])
    dst_v[pl.ds(EPW, 16)] = jnp.full((16,), 4 * N, dtype=jnp.int32)

    iota = lax.iota(jnp.int32, 16)

    def scan_body(i, offs):
        off_lo, off_hi = offs
        sl = pl.ds(i * 16, 16)
        gi = src_v[sl] * R + et_v[sl]
        d = dst_v[sl]
        bk = d // BROWS
        pk = gi * CAP + (d - bk * BROWS)
        for b in range(NW):
            m = bk == b
            cnt = plsc.all_reduce_population_count(m)
            if getattr(cnt, "ndim", 0):
                cnt_vec = cnt
            else:
                cnt_vec = jnp.full((16,), cnt, dtype=jnp.int32)
            ov = off_lo if b < 16 else off_hi
            off_b = jnp.minimum(ov[b % 16], CAP - 16)
            plsc.store_compressed(lists_v.at[pl.ds(b * CAP + off_b, 16)],
                                  pk, mask=m)
            upd = jnp.where(iota == (b % 16), cnt_vec, 0)
            if b < 16:
                off_lo = off_lo + upd
            else:
                off_hi = off_hi + upd
        return off_lo, off_hi

    zeros16 = jnp.zeros((16,), jnp.int32)
    off_lo, off_hi = lax.fori_loop(0, NSCAN, scan_body, (zeros16, zeros16))

    sent = jnp.full((16,), TRASH, dtype=jnp.int32)
    for b in range(NW):
        ov = off_lo if b < 16 else off_hi
        off_b = jnp.minimum(ov[b % 16], CAP - 16)
        lists_v[pl.ds(b * CAP + off_b, 16)] = sent
    nch_lo = jnp.minimum((off_lo + 15) // 16, CAP // 16)
    nch_hi = jnp.minimum((off_hi + 15) // 16, CAP // 16)
    cnt_v[pl.ds(0, 16)] = nch_lo
    cnt_v[pl.ds(16, 16)] = nch_hi
    cnt_v[pl.ds(32, 16)] = zeros16
    cnt_v[pl.ds(48, 16)] = zeros16

    pltpu.sync_copy(lists_v, packed_hbm.at[pl.ds(w * NW * CAP, NW * CAP)])
    pltpu.sync_copy(cnt_v, cnt_hbm.at[pl.ds(w * CNTW, CNTW)])


@functools.lru_cache(maxsize=None)
def _make_sc_scatter():
    return functools.partial(
        pl.kernel,
        mesh=_sc_mesh(),
        compiler_params=_SC_PARAMS,
        out_type=jax.ShapeDtypeStruct((N, D), jnp.float32),
        scratch_types=[
            pltpu.VMEM((NW * CNTW,), jnp.int32),
            pltpu.VMEM((CH,), jnp.int32),
            pltpu.VMEM((CH,), jnp.int32),
            pltpu.VMEM((CH, D), jnp.float32),
            pltpu.VMEM((ACC_R, D), jnp.float32),
            pltpu.SemaphoreType.DMA,
        ],
    )(_sc_scatter_body)


def _sc_scatter_body(hall_hbm, packed_hbm, cnt_hbm, out_hbm,
                     cnt_v, pk_v, gidx_v, rows_v, acc, sem):
    cid = lax.axis_index("c")
    sid = lax.axis_index("s")
    w = cid * NS + sid

    zf = jnp.zeros((16,), jnp.float32)

    def zero_body(i, _):
        for j in range(D // 16):
            acc[i, pl.ds(j * 16, 16)] = zf
        return 0

    lax.fori_loop(0, ACC_R, zero_body, 0)

    pltpu.sync_copy(cnt_hbm, cnt_v)

    def scanner_body(s2, _):
        nch = cnt_v[pl.ds(s2 * CNTW + w, 16)][0]
        base = (s2 * NW + w) * CAP

        def chunk_body(ch, _):
            pltpu.sync_copy(packed_hbm.at[pl.ds(base + ch * CH, CH)], pk_v)
            pk = pk_v[...]
            gi = pk // CAP
            dl = pk - gi * CAP
            gidx_v[...] = gi
            pltpu.async_copy(hall_hbm.at[gidx_v], rows_v, sem).wait()
            for l in range(CH):
                dlx = dl[l]
                for j in range(D // 16):
                    sl = pl.ds(j * 16, 16)
                    plsc.addupdate(acc.at[dlx, sl], rows_v[l, sl])
            return 0

        lax.fori_loop(0, nch, chunk_body, 0)
        return 0

    lax.fori_loop(0, NW, scanner_body, 0)

    rem = N - (NW - 1) * BROWS

    @pl.when(w < NW - 1)
    def _():
        pltpu.sync_copy(acc.at[pl.ds(0, BROWS)],
                        out_hbm.at[pl.ds(w * BROWS, BROWS)])

    @pl.when(w == NW - 1)
    def _():
        pltpu.sync_copy(acc.at[pl.ds(0, rem)],
                        out_hbm.at[pl.ds((NW - 1) * BROWS, rem)])


def kernel(feat, edge_index, etype, basis1, comp1, loop1, bias1,
           basis2, comp2, loop2, bias2, basis3, comp3, loop3, bias3,
           fc_w, fc_b):
    src = edge_index[0]
    dst = edge_index[1]

    packed, cnts = _make_sc_route()(src, dst, etype)

    hall = _tc_first(feat, basis1, comp1)
    agg = _make_sc_scatter()(hall.reshape(N * R, D), packed, cnts)

    h1, hall = _tc_mid(agg, feat, loop1, bias1.reshape(1, D), basis2, comp2)
    agg = _make_sc_scatter()(hall.reshape(N * R, D), packed, cnts)

    h2, hall = _tc_mid(agg, h1, loop2, bias2.reshape(1, D), basis3, comp3)
    agg = _make_sc_scatter()(hall.reshape(N * R, D), packed, cnts)

    return _tc_last(agg, h2, loop3, bias3.reshape(1, D), fc_w,
                    fc_b.reshape(1, D))


# speedup vs baseline: 1.5533x; 1.5533x over previous
import functools

import jax
import jax.numpy as jnp
from jax import lax
from jax.experimental import pallas as pl
from jax.experimental.pallas import tpu as pltpu
from jax.experimental.pallas import tpu_sc as plsc

N = 10000
E = 160000
R = 8
NB = 4
D = 256

NC = 2
NS = 16
NW = NC * NS
EPW = E // NW
NSCAN = EPW // 16 + 1
BROWS = 313
ACC_R = 320
TRASH = 316
CAP = 512
CH = 16
CNTW = 64

NBLK = 10
BLK = N // NBLK

_SC_PARAMS = pltpu.CompilerParams(use_tc_tiling_on_sc=False,
                                  needs_layout_passes=False)


def _rel_weight(basis_ref, comp_ref, r):
    def b16(x):
        return x.astype(jnp.bfloat16).astype(jnp.float32)

    w = b16(comp_ref[r, 0]) * b16(basis_ref[0])
    for b in range(1, NB):
        w = w + b16(comp_ref[r, b]) * b16(basis_ref[b])
    return w


def _tc_first_body(x_ref, basis_ref, comp_ref, hall_ref):
    r = pl.program_id(1)
    w = _rel_weight(basis_ref, comp_ref, r)
    hall_ref[...] = jnp.dot(x_ref[...].astype(jnp.bfloat16),
                            w.astype(jnp.bfloat16),
                            preferred_element_type=jnp.float32)


def _tc_first(x, basis, comp):
    return pl.pallas_call(
        _tc_first_body,
        grid=(NBLK, R),
        in_specs=[
            pl.BlockSpec((BLK, D), lambda n, r: (n, 0)),
            pl.BlockSpec((NB, D, D), lambda n, r: (0, 0, 0)),
            pl.BlockSpec(memory_space=pltpu.SMEM),
        ],
        out_specs=pl.BlockSpec((BLK, D), lambda n, r: (n, r)),
        out_shape=jax.ShapeDtypeStruct((N, R * D), jnp.float32),
    )(x, basis, comp)


def _tc_mid_body(agg_ref, x_ref, lw_ref, b_ref, basis_ref, comp_ref,
                 h_ref, hall_ref, hs_ref):
    r = pl.program_id(1)

    @pl.when(r == 0)
    def _():
        t = agg_ref[...] + jnp.dot(x_ref[...].astype(jnp.bfloat16),
                                   lw_ref[...].astype(jnp.bfloat16),
                                   preferred_element_type=jnp.float32)
        t = jnp.maximum(t + b_ref[...], 0.0)
        hs_ref[...] = t
        h_ref[...] = t

    w = _rel_weight(basis_ref, comp_ref, r)
    hall_ref[...] = jnp.dot(hs_ref[...].astype(jnp.bfloat16),
                            w.astype(jnp.bfloat16),
                            preferred_element_type=jnp.float32)


def _tc_mid(agg, x, loop_w, bias, basis, comp):
    return pl.pallas_call(
        _tc_mid_body,
        grid=(NBLK, R),
        in_specs=[
            pl.BlockSpec((BLK, D), lambda n, r: (n, 0)),
            pl.BlockSpec((BLK, D), lambda n, r: (n, 0)),
            pl.BlockSpec((D, D), lambda n, r: (0, 0)),
            pl.BlockSpec((1, D), lambda n, r: (0, 0)),
            pl.BlockSpec((NB, D, D), lambda n, r: (0, 0, 0)),
            pl.BlockSpec(memory_space=pltpu.SMEM),
        ],
        out_specs=[
            pl.BlockSpec((BLK, D), lambda n, r: (n, 0)),
            pl.BlockSpec((BLK, D), lambda n, r: (n, r)),
        ],
        out_shape=[
            jax.ShapeDtypeStruct((N, D), jnp.float32),
            jax.ShapeDtypeStruct((N, R * D), jnp.float32),
        ],
        scratch_shapes=[pltpu.VMEM((BLK, D), jnp.float32)],
    )(agg, x, loop_w, bias, basis, comp)


def _tc_last_body(agg_ref, x_ref, lw_ref, b_ref, fcw_ref, fcb_ref, out_ref):
    t = agg_ref[...] + jnp.dot(x_ref[...].astype(jnp.bfloat16),
                               lw_ref[...].astype(jnp.bfloat16),
                               preferred_element_type=jnp.float32)
    h = jnp.maximum(t + b_ref[...], 0.0)
    t = jnp.dot(h.astype(jnp.bfloat16), fcw_ref[...].astype(jnp.bfloat16),
                preferred_element_type=jnp.float32)
    t = t + fcb_ref[...]
    m = jnp.max(t, axis=1, keepdims=True)
    e = jnp.exp(t - m)
    out_ref[...] = e / jnp.sum(e, axis=1, keepdims=True)


def _tc_last(agg, x, loop_w, bias, fc_w, fc_b):
    return pl.pallas_call(
        _tc_last_body,
        grid=(NBLK,),
        in_specs=[
            pl.BlockSpec((BLK, D), lambda n: (n, 0)),
            pl.BlockSpec((BLK, D), lambda n: (n, 0)),
            pl.BlockSpec((D, D), lambda n: (0, 0)),
            pl.BlockSpec((1, D), lambda n: (0, 0)),
            pl.BlockSpec((D, D), lambda n: (0, 0)),
            pl.BlockSpec((1, D), lambda n: (0, 0)),
        ],
        out_specs=pl.BlockSpec((BLK, D), lambda n: (n, 0)),
        out_shape=jax.ShapeDtypeStruct((N, D), jnp.float32),
    )(agg, x, loop_w, bias, fc_w, fc_b)


def _sc_mesh():
    return plsc.VectorSubcoreMesh(core_axis_name="c", subcore_axis_name="s",
                                  num_cores=NC, num_subcores=NS)


@functools.lru_cache(maxsize=None)
def _make_sc_route():
    return functools.partial(
        pl.kernel,
        mesh=_sc_mesh(),
        compiler_params=_SC_PARAMS,
        out_type=(jax.ShapeDtypeStruct((NW * NW * CAP,), jnp.int32),
                  jax.ShapeDtypeStruct((NW * CNTW,), jnp.int32)),
        scratch_types=[
            pltpu.VMEM((EPW + 16,), jnp.int32),
            pltpu.VMEM((EPW + 16,), jnp.int32),
            pltpu.VMEM((EPW + 16,), jnp.int32),
            pltpu.VMEM((NW * CAP,), jnp.int32),
            pltpu.VMEM((CNTW,), jnp.int32),
        ],
    )(_sc_route_body)


def _sc_route_body(src_hbm, dst_hbm, et_hbm, packed_hbm, cnt_hbm,
                   src_v, dst_v, et_v, lists_v, cnt_v):
    cid = lax.axis_index("c")
    sid = lax.axis_index("s")
    w = cid * NS + sid

    pltpu.sync_copy(src_hbm.at[pl.ds(w * EPW, EPW)], src_v.at[pl.ds(0, EPW)])
    pltpu.sync_copy(dst_hbm.at[pl.ds(w * EPW, EPW)], dst_v.at[pl.ds(0, EPW)])
    pltpu.sync_copy(et_hbm.at[pl.ds(w * EPW, EPW)], et_v.at[pl.ds(0, EPW)])
    dst_v[pl.ds(EPW, 16)] = jnp.full((16,), 4 * N, dtype=jnp.int32)

    iota = lax.iota(jnp.int32, 16)

    def scan_body(i, offs):
        off_lo, off_hi = offs
        sl = pl.ds(i * 16, 16)
        gi = src_v[sl] * R + et_v[sl]
        d = dst_v[sl]
        bk = d // BROWS
        pk = gi * CAP + (d - bk * BROWS)
        for b in range(NW):
            m = bk == b
            cnt = plsc.all_reduce_population_count(m)
            if getattr(cnt, "ndim", 0):
                cnt_vec = cnt
            else:
                cnt_vec = jnp.full((16,), cnt, dtype=jnp.int32)
            ov = off_lo if b < 16 else off_hi
            off_b = jnp.minimum(ov[b % 16], CAP - 16)
            plsc.store_compressed(lists_v.at[pl.ds(b * CAP + off_b, 16)],
                                  pk, mask=m)
            upd = jnp.where(iota == (b % 16), cnt_vec, 0)
            if b < 16:
                off_lo = off_lo + upd
            else:
                off_hi = off_hi + upd
        return off_lo, off_hi

    zeros16 = jnp.zeros((16,), jnp.int32)
    off_lo, off_hi = lax.fori_loop(0, NSCAN, scan_body, (zeros16, zeros16))

    sent = jnp.full((16,), TRASH, dtype=jnp.int32)
    for b in range(NW):
        ov = off_lo if b < 16 else off_hi
        off_b = jnp.minimum(ov[b % 16], CAP - 16)
        lists_v[pl.ds(b * CAP + off_b, 16)] = sent
    nch_lo = jnp.minimum((off_lo + 15) // 16, CAP // 16)
    nch_hi = jnp.minimum((off_hi + 15) // 16, CAP // 16)
    cnt_v[pl.ds(0, 16)] = nch_lo
    cnt_v[pl.ds(16, 16)] = nch_hi
    cnt_v[pl.ds(32, 16)] = zeros16
    cnt_v[pl.ds(48, 16)] = zeros16

    pltpu.sync_copy(lists_v, packed_hbm.at[pl.ds(w * NW * CAP, NW * CAP)])
    pltpu.sync_copy(cnt_v, cnt_hbm.at[pl.ds(w * CNTW, CNTW)])


NBUF = 3
AMAX = NW * (CAP // CH) + 32


@functools.lru_cache(maxsize=None)
def _make_sc_scatter():
    return functools.partial(
        pl.kernel,
        mesh=_sc_mesh(),
        compiler_params=_SC_PARAMS,
        out_type=jax.ShapeDtypeStruct((N, D), jnp.float32),
        scratch_types=[
            pltpu.VMEM((NW * CNTW,), jnp.int32),
            pltpu.VMEM((AMAX,), jnp.int32),
            pltpu.VMEM((NBUF, CH), jnp.int32),
            pltpu.VMEM((NBUF, CH), jnp.int32),
            pltpu.VMEM((NBUF, CH, D), jnp.float32),
            pltpu.VMEM((ACC_R, D), jnp.float32),
            pltpu.SemaphoreType.DMA,
            pltpu.SemaphoreType.DMA,
            pltpu.SemaphoreType.DMA,
            pltpu.SemaphoreType.DMA,
            pltpu.SemaphoreType.DMA,
            pltpu.SemaphoreType.DMA,
        ],
    )(_sc_scatter_body)


def _sc_scatter_body(hall_hbm, packed_hbm, cnt_hbm, out_hbm,
                     cnt_v, addr_v, pkb, gib, rowsb, acc,
                     ps0, ps1, ps2, gs0, gs1, gs2):
    cid = lax.axis_index("c")
    sid = lax.axis_index("s")
    w = cid * NS + sid
    psem = [ps0, ps1, ps2]
    gsem = [gs0, gs1, gs2]

    zf = jnp.zeros((16,), jnp.float32)

    def zero_body(i, _):
        for j in range(D // 16):
            acc[i, pl.ds(j * 16, 16)] = zf
        return 0

    lax.fori_loop(0, ACC_R, zero_body, 0)

    pltpu.sync_copy(cnt_hbm, cnt_v)

    iota = lax.iota(jnp.int32, 16)
    tot = 0
    for s2 in range(NW):
        nch = cnt_v[pl.ds(s2 * CNTW + w, 16)][0]
        base = (s2 * NW + w) * CAP
        n1 = jnp.minimum(nch, 16)
        plsc.store_compressed(addr_v.at[pl.ds(tot, 16)],
                              base + iota * CH, mask=iota < n1)
        plsc.store_compressed(addr_v.at[pl.ds(tot + n1, 16)],
                              base + (iota + 16) * CH, mask=iota < nch - n1)
        tot = tot + nch

    def fetch(k, b):
        @pl.when(k < tot)
        def _():
            a = pl.multiple_of(addr_v[pl.ds(k, 16)][0], CH)
            pltpu.async_copy(packed_hbm.at[pl.ds(a, CH)], pkb.at[b],
                             psem[b])

    def gather(k, b):
        @pl.when((k >= 0) & (k < tot))
        def _():
            pltpu.make_async_copy(packed_hbm.at[pl.ds(0, CH)], pkb.at[b],
                                  psem[b]).wait()
            gib[b] = pkb[b] >> 9
            pltpu.async_copy(hall_hbm.at[gib.at[b]], rowsb.at[b], gsem[b])

    def accum(k, b):
        @pl.when((k >= 0) & (k < tot))
        def _():
            pltpu.make_async_copy(hall_hbm.at[gib.at[b]],
                                  rowsb.at[b], gsem[b]).wait()
            dl = pkb[b] & (CAP - 1)
            for l in range(CH):
                dlx = dl[l]
                vals = [rowsb[b, l, pl.ds(j * 16, 16)]
                        for j in range(D // 16)]
                for j in range(D // 16):
                    plsc.addupdate(acc.at[dlx, pl.ds(j * 16, 16)], vals[j])

    def tri_body(t, _):
        for bb in range(NBUF):
            k = t * NBUF + bb
            accum(k - NBUF, bb)
            gather(k - 1, (bb + NBUF - 1) % NBUF)
            fetch(k, bb)
        return 0

    ntri = (tot + NBUF + NBUF - 1) // NBUF
    lax.fori_loop(0, ntri, tri_body, 0)

    rem = N - (NW - 1) * BROWS

    @pl.when(w < NW - 1)
    def _():
        pltpu.sync_copy(acc.at[pl.ds(0, BROWS)],
                        out_hbm.at[pl.ds(w * BROWS, BROWS)])

    @pl.when(w == NW - 1)
    def _():
        pltpu.sync_copy(acc.at[pl.ds(0, rem)],
                        out_hbm.at[pl.ds((NW - 1) * BROWS, rem)])


def kernel(feat, edge_index, etype, basis1, comp1, loop1, bias1,
           basis2, comp2, loop2, bias2, basis3, comp3, loop3, bias3,
           fc_w, fc_b):
    src = edge_index[0]
    dst = edge_index[1]

    packed, cnts = _make_sc_route()(src, dst, etype)

    hall = _tc_first(feat, basis1, comp1)
    agg = _make_sc_scatter()(hall.reshape(N * R, D), packed, cnts)

    h1, hall = _tc_mid(agg, feat, loop1, bias1.reshape(1, D), basis2, comp2)
    agg = _make_sc_scatter()(hall.reshape(N * R, D), packed, cnts)

    h2, hall = _tc_mid(agg, h1, loop2, bias2.reshape(1, D), basis3, comp3)
    agg = _make_sc_scatter()(hall.reshape(N * R, D), packed, cnts)

    return _tc_last(agg, h2, loop3, bias3.reshape(1, D), fc_w,
                    fc_b.reshape(1, D))
